# baseline (speedup 1.0000x reference)
import functools

import jax
import jax.numpy as jnp
from jax import lax
from jax.experimental import pallas as pl
from jax.experimental.pallas import tpu as pltpu
from jax.experimental.pallas import tpu_sc as plsc

B = 4
N0, N1, N2 = 40000, 10000, 2500
L = 9
CIN, C0, C1 = 3, 16, 32
LAT = 128
K = 3
NW = 32
N0P, N1P, N2P = 40960, 10240, 2560
M0, M1 = N0P * L, N1P * L


def _mesh():
    return plsc.VectorSubcoreMesh(core_axis_name="c", subcore_axis_name="s")


def _wid():
    return lax.axis_index("s") * 2 + lax.axis_index("c")


def _sc_gather(table, idxf, d, chunk, iters, s, lin=False):
    m = idxf.shape[0]
    per_w = chunk * iters
    nsuper = iters // s
    sc = s * chunk
    assert per_w * NW == m and chunk % 8 == 0 and chunk <= 128
    assert nsuper * s == iters and nsuper % 2 == 0

    @functools.partial(
        pl.kernel,
        out_type=jax.ShapeDtypeStruct((m, d), jnp.float32),
        mesh=_mesh(),
        compiler_params=pltpu.CompilerParams(use_tc_tiling_on_sc=False) if lin else None,
        scratch_types=[
            pltpu.VMEM((per_w,), jnp.int32),
            pltpu.VMEM((2, sc, d), jnp.float32),
            pltpu.SemaphoreType.DMA,
            pltpu.SemaphoreType.DMA,
            pltpu.SemaphoreType.DMA,
            pltpu.SemaphoreType.DMA,
        ],
    )
    def gk(table_hbm, idx_hbm, out_hbm, idx_v, rows_v, g0, g1, w0, w1):
        base = _wid() * per_w
        pltpu.sync_copy(idx_hbm.at[pl.ds(base, per_w)], idx_v)
        gsem = (g0, g1)
        wsem = (w0, w1)

        def super_body(jj, c):
            for b in (0, 1):
                j = 2 * jj + b

                @pl.when(j >= 2)
                def _():
                    pltpu.make_async_copy(
                        rows_v.at[b], out_hbm.at[pl.ds(0, sc)], wsem[b]
                    ).wait()

                descs = []
                for t in range(s):
                    ix = idx_v.at[pl.ds((j * s + t) * chunk, chunk)]
                    descs.append(pltpu.async_copy(
                        table_hbm.at[ix],
                        rows_v.at[b, pl.ds(t * chunk, chunk)],
                        gsem[b]))
                for dsc in descs:
                    dsc.wait()
                pltpu.async_copy(
                    rows_v.at[b], out_hbm.at[pl.ds(base + j * sc, sc)], wsem[b])
            return c

        lax.fori_loop(0, nsuper // 2, super_body, 0)
        for b in (0, 1):
            pltpu.make_async_copy(
                rows_v.at[b], out_hbm.at[pl.ds(0, sc)], wsem[b]).wait()

    return gk(table, idxf)


def _sc_pool(table, colf, val16, nout, d, ochunk, iters, s, lin=False):
    mc = colf.shape[0]
    cchunk = ochunk * K
    per_w = ochunk * iters
    per_wc = cchunk * iters
    nsuper = iters // s
    so = s * ochunk
    sc = s * cchunk
    assert per_w * NW == nout and per_wc * NW == mc
    assert cchunk % 8 == 0 and cchunk <= 128 and d % 16 == 0
    assert nsuper * s == iters and nsuper % 2 == 0

    @functools.partial(
        pl.kernel,
        out_type=jax.ShapeDtypeStruct((nout, d), jnp.float32),
        mesh=_mesh(),
        compiler_params=pltpu.CompilerParams(use_tc_tiling_on_sc=False) if lin else None,
        scratch_types=[
            pltpu.VMEM((per_wc,), jnp.int32),
            pltpu.VMEM((2, sc, d), jnp.float32),
            pltpu.VMEM((2, sc, 16), jnp.float32),
            pltpu.VMEM((2, so, d), jnp.float32),
            pltpu.SemaphoreType.DMA,
            pltpu.SemaphoreType.DMA,
            pltpu.SemaphoreType.DMA,
            pltpu.SemaphoreType.DMA,
            pltpu.SemaphoreType.DMA,
            pltpu.SemaphoreType.DMA,
        ],
    )
    def pk(table_hbm, col_hbm, val_hbm, out_hbm,
           col_v, rows_v, val_v, out_v, g0, g1, v0, v1, w0, w1):
        rbase = _wid() * per_w
        cbase = rbase * K
        gsem = (g0, g1)
        vsem = (v0, v1)
        wsem = (w0, w1)
        pltpu.sync_copy(col_hbm.at[pl.ds(cbase, per_wc)], col_v)

        def fire(j, buf):
            for t in range(s):
                ix = col_v.at[pl.ds((j * s + t) * cchunk, cchunk)]
                pltpu.async_copy(
                    table_hbm.at[ix],
                    rows_v.at[buf, pl.ds(t * cchunk, cchunk)], gsem[buf])
            pltpu.async_copy(
                val_hbm.at[pl.ds(cbase + j * sc, sc)], val_v.at[buf], vsem[buf])

        fire(0, 0)

        def super_body(jj, c):
            for b in (0, 1):
                j = 2 * jj + b

                @pl.when(j + 1 < nsuper)
                def _():
                    fire(j + 1, 1 - b)

                for t in range(s):
                    pltpu.make_async_copy(
                        table_hbm.at[pl.ds(0, cchunk)],
                        rows_v.at[b, pl.ds(t * cchunk, cchunk)], gsem[b]).wait()
                pltpu.make_async_copy(
                    val_hbm.at[pl.ds(0, sc)], val_v.at[b], vsem[b]).wait()

                @pl.when(j >= 2)
                def _():
                    pltpu.make_async_copy(
                        out_v.at[b], out_hbm.at[pl.ds(0, so)], wsem[b]).wait()

                def rbody(r, c2):
                    r3 = r * K
                    for db in range(d // 16):
                        sl = pl.ds(db * 16, 16)
                        acc = val_v[b, r3, :] * rows_v[b, r3, sl]
                        acc = acc + val_v[b, r3 + 1, :] * rows_v[b, r3 + 1, sl]
                        acc = acc + val_v[b, r3 + 2, :] * rows_v[b, r3 + 2, sl]
                        out_v[b, r, sl] = acc
                    return c2

                lax.fori_loop(0, so, rbody, 0)
                pltpu.async_copy(
                    out_v.at[b], out_hbm.at[pl.ds(rbase + j * so, so)], wsem[b])
            return c

        lax.fori_loop(0, nsuper // 2, super_body, 0)
        for b in (0, 1):
            pltpu.make_async_copy(
                out_v.at[b], out_hbm.at[pl.ds(0, so)], wsem[b]).wait()

    return pk(table, colf, val16)


def _sc_sum9(table, idxf, d, bias, elu, chunk, iters, s, lin=True):
    m = idxf.shape[0]
    per_w = chunk * iters
    nsuper = iters // s
    sg = s * chunk
    og = sg // L
    per_wo = per_w // L
    nout = m // L
    assert per_w * NW == m and chunk % 8 == 0 and chunk <= 128
    assert nsuper * s == iters and nsuper % 2 == 0 and sg % L == 0
    assert d % 16 == 0

    @functools.partial(
        pl.kernel,
        out_type=jax.ShapeDtypeStruct((nout, d), jnp.float32),
        mesh=_mesh(),
        compiler_params=pltpu.CompilerParams(use_tc_tiling_on_sc=False) if lin else None,
        scratch_types=[
            pltpu.VMEM((per_w,), jnp.int32),
            pltpu.VMEM((2, sg, d), jnp.float32),
            pltpu.VMEM((2, og, d), jnp.float32),
            pltpu.VMEM((d,), jnp.float32),
            pltpu.SemaphoreType.DMA,
            pltpu.SemaphoreType.DMA,
            pltpu.SemaphoreType.DMA,
            pltpu.SemaphoreType.DMA,
        ],
    )
    def sk(table_hbm, idx_hbm, bias_hbm, out_hbm,
           idx_v, rows_v, out_v, bias_v, g0, g1, w0, w1):
        base = _wid() * per_w
        rbase = _wid() * per_wo
        gsem = (g0, g1)
        wsem = (w0, w1)
        pltpu.sync_copy(idx_hbm.at[pl.ds(base, per_w)], idx_v)
        pltpu.sync_copy(bias_hbm, bias_v)

        def fire(j, buf):
            for t in range(s):
                ix = idx_v.at[pl.ds((j * s + t) * chunk, chunk)]
                pltpu.async_copy(
                    table_hbm.at[ix],
                    rows_v.at[buf, pl.ds(t * chunk, chunk)], gsem[buf])

        fire(0, 0)

        def super_body(jj, c):
            for b in (0, 1):
                j = 2 * jj + b

                @pl.when(j + 1 < nsuper)
                def _():
                    fire(j + 1, 1 - b)

                for t in range(s):
                    pltpu.make_async_copy(
                        table_hbm.at[pl.ds(0, chunk)],
                        rows_v.at[b, pl.ds(t * chunk, chunk)], gsem[b]).wait()

                @pl.when(j >= 2)
                def _():
                    pltpu.make_async_copy(
                        out_v.at[b], out_hbm.at[pl.ds(0, og)], wsem[b]).wait()

                def rbody(r, c2):
                    r9 = r * L
                    for db in range(d // 16):
                        sl = pl.ds(db * 16, 16)
                        acc = rows_v[b, r9, sl]
                        for q in range(1, L):
                            acc = acc + rows_v[b, r9 + q, sl]
                        acc = acc + bias_v[sl]
                        if elu:
                            acc = jnp.where(
                                acc > 0, acc,
                                jnp.exp(jnp.minimum(acc, 0.0)) - 1.0)
                        out_v[b, r, sl] = acc
                    return c2

                lax.fori_loop(0, og, rbody, 0)
                pltpu.async_copy(
                    out_v.at[b], out_hbm.at[pl.ds(rbase + j * og, og)], wsem[b])
            return c

        lax.fori_loop(0, nsuper // 2, super_body, 0)
        for b in (0, 1):
            pltpu.make_async_copy(
                out_v.at[b], out_hbm.at[pl.ds(0, og)], wsem[b]).wait()

    return sk(table, idxf, bias)


def _tc_mm_stack(a, w2, blk):
    n, kd = a.shape
    co = w2.shape[1]
    nblk = n // blk

    def body(a_ref, w_ref, o_ref):
        l = pl.program_id(1)
        o_ref[...] = jnp.dot(a_ref[...], w_ref[pl.ds(l * kd, kd), :],
                             preferred_element_type=jnp.float32)

    return pl.pallas_call(
        body,
        grid=(nblk, L),
        in_specs=[
            pl.BlockSpec((blk, kd), lambda i, l: (i, 0)),
            pl.BlockSpec((L * kd, co), lambda i, l: (0, 0)),
        ],
        out_specs=pl.BlockSpec((blk, co), lambda i, l: (l * nblk + i, 0)),
        out_shape=jax.ShapeDtypeStruct((L * n, co), jnp.float32),
    )(a, w2)


def _tc_mm(a, w, bias, act, blk):
    n, kd = a.shape
    co = w.shape[1]
    assert n % blk == 0

    def body(a_ref, w_ref, b_ref, o_ref):
        z = jnp.dot(a_ref[...], w_ref[...], preferred_element_type=jnp.float32)
        z = z + b_ref[...]
        if act:
            z = jnp.where(z > 0, z, jnp.exp(jnp.minimum(z, 0.0)) - 1.0)
        o_ref[...] = z

    return pl.pallas_call(
        body,
        grid=(n // blk,),
        in_specs=[
            pl.BlockSpec((blk, kd), lambda i: (i, 0)),
            pl.BlockSpec((kd, co), lambda i: (0, 0)),
            pl.BlockSpec((1, co), lambda i: (0, 0)),
        ],
        out_specs=pl.BlockSpec((blk, co), lambda i: (i, 0)),
        out_shape=jax.ShapeDtypeStruct((n, co), jnp.float32),
    )(a, w, bias.reshape(1, co))


def _lat_enc(a8, wl, bl):
    kc = 3200
    g = a8.shape[1] // kc

    def body(a_ref, w_ref, b_ref, o_ref):
        @pl.when(pl.program_id(0) == 0)
        def _():
            o_ref[...] = jnp.broadcast_to(b_ref[...], o_ref.shape)

        o_ref[...] += lax.dot_general(
            a_ref[...], w_ref[...], (((1,), (1,)), ((), ())),
            preferred_element_type=jnp.float32)

    return pl.pallas_call(
        body,
        grid=(g,),
        in_specs=[
            pl.BlockSpec((8, kc), lambda i: (0, i)),
            pl.BlockSpec((LAT, kc), lambda i: (0, i)),
            pl.BlockSpec((1, LAT), lambda i: (0, 0)),
        ],
        out_specs=pl.BlockSpec((8, LAT), lambda i: (0, 0)),
        out_shape=jax.ShapeDtypeStruct((8, LAT), jnp.float32),
    )(a8, wl, bl.reshape(1, LAT))


def _lat_dec(z8, wd, bd):
    rc = 3200
    g = wd.shape[0] // rc

    def body(z_ref, w_ref, b_ref, o_ref):
        o_ref[...] = lax.dot_general(
            z_ref[...], w_ref[...], (((1,), (1,)), ((), ())),
            preferred_element_type=jnp.float32) + b_ref[...]

    return pl.pallas_call(
        body,
        grid=(g,),
        in_specs=[
            pl.BlockSpec((8, LAT), lambda i: (0, 0)),
            pl.BlockSpec((rc, LAT), lambda i: (i, 0)),
            pl.BlockSpec((1, rc), lambda i: (0, i)),
        ],
        out_specs=pl.BlockSpec((8, rc), lambda i: (0, i)),
        out_shape=jax.ShapeDtypeStruct((8, wd.shape[0]), jnp.float32),
    )(z8, wd, bd.reshape(1, wd.shape[0]))


def _expand_w(w, cin, cout, ctab):
    t = jnp.transpose(w.reshape(cout, L, cin), (1, 2, 0))
    j = jnp.arange(ctab)
    tl = t[:, j % cin, :]
    onehot = ((j[:, None] // cin) == jnp.arange(B)[None, :]) & (j < B * cin)[:, None]
    w4 = tl[:, :, None, :] * onehot[None, :, :, None].astype(w.dtype)
    return w4.reshape(L * ctab, B * cout)


def _padc(w, to):
    return jnp.pad(w, ((0, 0), (0, to - w.shape[1])))


def kernel(x, idx0, idx1, dt0_row, dt0_col, dt0_val, dt1_row, dt1_col, dt1_val,
           ut0_row, ut0_col, ut0_val, ut1_row, ut1_col, ut1_val,
           W_en0, b_en0, W_en1, b_en1, W_enl, b_enl, W_del, b_del,
           W_de1, b_de1, W_de2, b_de2, W_f, b_f):
    f32 = jnp.float32
    xt = jnp.transpose(x, (1, 0, 2)).reshape(N0, B * CIN)
    xt = jnp.pad(xt, ((0, 0), (0, 16 - B * CIN)))
    idx0f = jnp.pad(idx0.astype(jnp.int32).reshape(-1), (0, M0 - N0 * L))
    idx1f = jnp.pad(idx1.astype(jnp.int32).reshape(-1), (0, M1 - N1 * L))

    def colval(col, val, mp):
        c = jnp.pad(col.astype(jnp.int32), (0, mp - col.shape[0]))
        v = jnp.pad(val.astype(f32), (0, mp - val.shape[0]))
        return c, jnp.broadcast_to(v[:, None], (mp, 16))

    dt0c, dt0v = colval(dt0_col, dt0_val, N1P * K)
    dt1c, dt1v = colval(dt1_col, dt1_val, N2P * K)
    ut1c, ut1v = colval(ut1_col, ut1_val, N1P * K)
    ut0c, ut0v = colval(ut0_col, ut0_val, N0P * K)

    def pad128(b):
        return jnp.pad(jnp.tile(b, B), (0, 128 - B * b.shape[0]))

    g1 = _sc_gather(xt, idx0f, 16, 128, 90, 3, lin=True).reshape(N0P, L * 16)
    h1 = _tc_mm(g1, _expand_w(W_en0, CIN, C0, 16), jnp.tile(b_en0, B), True, 256)
    p1 = _sc_pool(h1, dt0c, dt0v, N1P, 64, 32, 10, 1, lin=True)
    idx1f9 = idx1f + (jnp.arange(M1, dtype=jnp.int32) % L) * N1P
    t2 = _tc_mm_stack(p1, _expand_w(W_en1, C0, C1, 64), 256)
    h2 = _sc_sum9(t2, idx1f9, 128, jnp.tile(b_en1, B), True, 96, 30, 3, lin=False)
    p2 = _sc_pool(h2, dt1c, dt1v, N2P, 128, 40, 2, 1)

    p2t = jnp.transpose(p2[:N2].reshape(N2, B, C1), (1, 0, 2)).reshape(B, N2 * C1)
    z8 = _lat_enc(jnp.pad(p2t, ((0, 4), (0, 0))), W_enl, b_enl)
    d8 = _lat_dec(z8, W_del, b_del)
    d0 = jnp.transpose(d8[:B].reshape(B, N2, C1), (1, 0, 2)).reshape(N2, B * C1)

    p3 = _sc_pool(d0, ut1c, ut1v, N1P, 128, 32, 10, 1)
    t3 = _tc_mm_stack(p3, _expand_w(W_de1, C1, C1, 128), 256)
    h4 = _sc_sum9(t3, idx1f9, 128, jnp.tile(b_de1, B), True, 96, 30, 3, lin=False)
    p4 = _sc_pool(h4, ut0c, ut0v, N0P, 128, 32, 40, 2)
    idx0f4 = idx0f + (jnp.arange(M0, dtype=jnp.int32) % L) * N0P
    t4 = _tc_mm_stack(p4, _expand_w(W_de2, C1, C0, 128), 256)
    h5 = _sc_sum9(t4, idx0f4, 64, jnp.tile(b_de2, B), True, 96, 120, 3)
    t5 = _tc_mm_stack(h5, _padc(_expand_w(W_f, C0, CIN, 64), 16), 256)
    bf = jnp.pad(jnp.tile(b_f, B), (0, 4))
    outp = _sc_sum9(t5, idx0f4, 16, bf, False, 96, 120, 3)

    return jnp.transpose(outp[:N0, :B * CIN].reshape(N0, B, CIN), (1, 0, 2))


# speedup vs baseline: 1.1473x; 1.1473x over previous
import functools

import jax
import jax.numpy as jnp
from jax import lax
from jax.experimental import pallas as pl
from jax.experimental.pallas import tpu as pltpu
from jax.experimental.pallas import tpu_sc as plsc

B = 4
N0, N1, N2 = 40000, 10000, 2500
L = 9
CIN, C0, C1 = 3, 16, 32
LAT = 128
K = 3
NW = 32
N0P, N1P, N2P = 40960, 10240, 2560
M0, M1 = N0P * L, N1P * L


def _mesh():
    return plsc.VectorSubcoreMesh(core_axis_name="c", subcore_axis_name="s")


def _wid():
    return lax.axis_index("s") * 2 + lax.axis_index("c")


def _sc_gather(table, idxf, d, chunk, iters, s, lin=False):
    m = idxf.shape[0]
    per_w = chunk * iters
    nsuper = iters // s
    sc = s * chunk
    assert per_w * NW == m and chunk % 8 == 0 and chunk <= 128
    assert nsuper * s == iters and nsuper % 2 == 0

    @functools.partial(
        pl.kernel,
        out_type=jax.ShapeDtypeStruct((m, d), jnp.float32),
        mesh=_mesh(),
        compiler_params=pltpu.CompilerParams(use_tc_tiling_on_sc=False) if lin else None,
        scratch_types=[
            pltpu.VMEM((per_w,), jnp.int32),
            pltpu.VMEM((2, sc, d), jnp.float32),
            pltpu.SemaphoreType.DMA,
            pltpu.SemaphoreType.DMA,
            pltpu.SemaphoreType.DMA,
            pltpu.SemaphoreType.DMA,
        ],
    )
    def gk(table_hbm, idx_hbm, out_hbm, idx_v, rows_v, g0, g1, w0, w1):
        base = _wid() * per_w
        pltpu.sync_copy(idx_hbm.at[pl.ds(base, per_w)], idx_v)
        gsem = (g0, g1)
        wsem = (w0, w1)

        def super_body(jj, c):
            for b in (0, 1):
                j = 2 * jj + b

                @pl.when(j >= 2)
                def _():
                    pltpu.make_async_copy(
                        rows_v.at[b], out_hbm.at[pl.ds(0, sc)], wsem[b]
                    ).wait()

                descs = []
                for t in range(s):
                    ix = idx_v.at[pl.ds((j * s + t) * chunk, chunk)]
                    descs.append(pltpu.async_copy(
                        table_hbm.at[ix],
                        rows_v.at[b, pl.ds(t * chunk, chunk)],
                        gsem[b]))
                for dsc in descs:
                    dsc.wait()
                pltpu.async_copy(
                    rows_v.at[b], out_hbm.at[pl.ds(base + j * sc, sc)], wsem[b])
            return c

        lax.fori_loop(0, nsuper // 2, super_body, 0)
        for b in (0, 1):
            pltpu.make_async_copy(
                rows_v.at[b], out_hbm.at[pl.ds(0, sc)], wsem[b]).wait()

    return gk(table, idxf)


def _sc_pool(table, colf, val16, nout, d, ochunk, iters, s, lin=False):
    mc = colf.shape[0]
    cchunk = ochunk * K
    per_w = ochunk * iters
    per_wc = cchunk * iters
    nsuper = iters // s
    so = s * ochunk
    sc = s * cchunk
    assert per_w * NW == nout and per_wc * NW == mc
    assert cchunk % 8 == 0 and cchunk <= 128 and d % 16 == 0
    assert nsuper * s == iters and nsuper % 2 == 0

    @functools.partial(
        pl.kernel,
        out_type=jax.ShapeDtypeStruct((nout, d), jnp.float32),
        mesh=_mesh(),
        compiler_params=pltpu.CompilerParams(use_tc_tiling_on_sc=False) if lin else None,
        scratch_types=[
            pltpu.VMEM((per_wc,), jnp.int32),
            pltpu.VMEM((2, sc, d), jnp.float32),
            pltpu.VMEM((2, sc, 16), jnp.float32),
            pltpu.VMEM((2, so, d), jnp.float32),
            pltpu.SemaphoreType.DMA,
            pltpu.SemaphoreType.DMA,
            pltpu.SemaphoreType.DMA,
            pltpu.SemaphoreType.DMA,
            pltpu.SemaphoreType.DMA,
            pltpu.SemaphoreType.DMA,
        ],
    )
    def pk(table_hbm, col_hbm, val_hbm, out_hbm,
           col_v, rows_v, val_v, out_v, g0, g1, v0, v1, w0, w1):
        rbase = _wid() * per_w
        cbase = rbase * K
        gsem = (g0, g1)
        vsem = (v0, v1)
        wsem = (w0, w1)
        pltpu.sync_copy(col_hbm.at[pl.ds(cbase, per_wc)], col_v)

        def fire(j, buf):
            for t in range(s):
                ix = col_v.at[pl.ds((j * s + t) * cchunk, cchunk)]
                pltpu.async_copy(
                    table_hbm.at[ix],
                    rows_v.at[buf, pl.ds(t * cchunk, cchunk)], gsem[buf])
            pltpu.async_copy(
                val_hbm.at[pl.ds(cbase + j * sc, sc)], val_v.at[buf], vsem[buf])

        fire(0, 0)

        def super_body(jj, c):
            for b in (0, 1):
                j = 2 * jj + b

                @pl.when(j + 1 < nsuper)
                def _():
                    fire(j + 1, 1 - b)

                for t in range(s):
                    pltpu.make_async_copy(
                        table_hbm.at[pl.ds(0, cchunk)],
                        rows_v.at[b, pl.ds(t * cchunk, cchunk)], gsem[b]).wait()
                pltpu.make_async_copy(
                    val_hbm.at[pl.ds(0, sc)], val_v.at[b], vsem[b]).wait()

                @pl.when(j >= 2)
                def _():
                    pltpu.make_async_copy(
                        out_v.at[b], out_hbm.at[pl.ds(0, so)], wsem[b]).wait()

                def rbody(r, c2):
                    r3 = r * K
                    for db in range(d // 16):
                        sl = pl.ds(db * 16, 16)
                        acc = val_v[b, r3, :] * rows_v[b, r3, sl]
                        acc = acc + val_v[b, r3 + 1, :] * rows_v[b, r3 + 1, sl]
                        acc = acc + val_v[b, r3 + 2, :] * rows_v[b, r3 + 2, sl]
                        out_v[b, r, sl] = acc
                    return c2

                lax.fori_loop(0, so, rbody, 0)
                pltpu.async_copy(
                    out_v.at[b], out_hbm.at[pl.ds(rbase + j * so, so)], wsem[b])
            return c

        lax.fori_loop(0, nsuper // 2, super_body, 0)
        for b in (0, 1):
            pltpu.make_async_copy(
                out_v.at[b], out_hbm.at[pl.ds(0, so)], wsem[b]).wait()

    return pk(table, colf, val16)


def _sc_sum9(table, idxf, d, bias, elu, chunk, iters, s, lin=True):
    m = idxf.shape[0]
    per_w = chunk * iters
    nsuper = iters // s
    sg = s * chunk
    og = sg // L
    per_wo = per_w // L
    nout = m // L
    assert per_w * NW == m and chunk % 8 == 0 and chunk <= 128
    assert nsuper * s == iters and nsuper % 2 == 0 and sg % L == 0
    assert d % 16 == 0

    @functools.partial(
        pl.kernel,
        out_type=jax.ShapeDtypeStruct((nout, d), jnp.float32),
        mesh=_mesh(),
        compiler_params=pltpu.CompilerParams(use_tc_tiling_on_sc=False) if lin else None,
        scratch_types=[
            pltpu.VMEM((per_w,), jnp.int32),
            pltpu.VMEM((2, sg, d), jnp.float32),
            pltpu.VMEM((2, og, d), jnp.float32),
            pltpu.VMEM((d,), jnp.float32),
            pltpu.SemaphoreType.DMA,
            pltpu.SemaphoreType.DMA,
            pltpu.SemaphoreType.DMA,
            pltpu.SemaphoreType.DMA,
        ],
    )
    def sk(table_hbm, idx_hbm, bias_hbm, out_hbm,
           idx_v, rows_v, out_v, bias_v, g0, g1, w0, w1):
        base = _wid() * per_w
        rbase = _wid() * per_wo
        gsem = (g0, g1)
        wsem = (w0, w1)
        pltpu.sync_copy(idx_hbm.at[pl.ds(base, per_w)], idx_v)
        pltpu.sync_copy(bias_hbm, bias_v)

        def fire(j, buf):
            for t in range(s):
                ix = idx_v.at[pl.ds((j * s + t) * chunk, chunk)]
                pltpu.async_copy(
                    table_hbm.at[ix],
                    rows_v.at[buf, pl.ds(t * chunk, chunk)], gsem[buf])

        fire(0, 0)

        def super_body(jj, c):
            for b in (0, 1):
                j = 2 * jj + b

                @pl.when(j + 1 < nsuper)
                def _():
                    fire(j + 1, 1 - b)

                for t in range(s):
                    pltpu.make_async_copy(
                        table_hbm.at[pl.ds(0, chunk)],
                        rows_v.at[b, pl.ds(t * chunk, chunk)], gsem[b]).wait()

                @pl.when(j >= 2)
                def _():
                    pltpu.make_async_copy(
                        out_v.at[b], out_hbm.at[pl.ds(0, og)], wsem[b]).wait()

                def rbody(r, c2):
                    r9 = r * L
                    for db in range(d // 16):
                        sl = pl.ds(db * 16, 16)
                        acc = rows_v[b, r9, sl]
                        for q in range(1, L):
                            acc = acc + rows_v[b, r9 + q, sl]
                        acc = acc + bias_v[sl]
                        if elu:
                            acc = jnp.where(
                                acc > 0, acc,
                                jnp.exp(jnp.minimum(acc, 0.0)) - 1.0)
                        out_v[b, r, sl] = acc
                    return c2

                lax.fori_loop(0, og, rbody, 0)
                pltpu.async_copy(
                    out_v.at[b], out_hbm.at[pl.ds(rbase + j * og, og)], wsem[b])
            return c

        lax.fori_loop(0, nsuper // 2, super_body, 0)
        for b in (0, 1):
            pltpu.make_async_copy(
                out_v.at[b], out_hbm.at[pl.ds(0, og)], wsem[b]).wait()

    return sk(table, idxf, bias)


def _tc_mm_stack(a, w2, blk):
    n, kd = a.shape
    co = w2.shape[1]
    nblk = n // blk

    def body(a_ref, w_ref, o_ref):
        l = pl.program_id(1)
        o_ref[...] = jnp.dot(a_ref[...], w_ref[pl.ds(l * kd, kd), :],
                             preferred_element_type=jnp.float32)

    return pl.pallas_call(
        body,
        grid=(nblk, L),
        in_specs=[
            pl.BlockSpec((blk, kd), lambda i, l: (i, 0)),
            pl.BlockSpec((L * kd, co), lambda i, l: (0, 0)),
        ],
        out_specs=pl.BlockSpec((blk, co), lambda i, l: (l * nblk + i, 0)),
        out_shape=jax.ShapeDtypeStruct((L * n, co), jnp.float32),
    )(a, w2)


def _tc_mm(a, w, bias, act, blk):
    n, kd = a.shape
    co = w.shape[1]
    assert n % blk == 0

    def body(a_ref, w_ref, b_ref, o_ref):
        z = jnp.dot(a_ref[...], w_ref[...], preferred_element_type=jnp.float32)
        z = z + b_ref[...]
        if act:
            z = jnp.where(z > 0, z, jnp.exp(jnp.minimum(z, 0.0)) - 1.0)
        o_ref[...] = z

    return pl.pallas_call(
        body,
        grid=(n // blk,),
        in_specs=[
            pl.BlockSpec((blk, kd), lambda i: (i, 0)),
            pl.BlockSpec((kd, co), lambda i: (0, 0)),
            pl.BlockSpec((1, co), lambda i: (0, 0)),
        ],
        out_specs=pl.BlockSpec((blk, co), lambda i: (i, 0)),
        out_shape=jax.ShapeDtypeStruct((n, co), jnp.float32),
    )(a, w, bias.reshape(1, co))


def _lat_enc(a8, wl, bl):
    kc = 3200
    g = a8.shape[1] // kc

    def body(a_ref, w_ref, b_ref, o_ref):
        @pl.when(pl.program_id(0) == 0)
        def _():
            o_ref[...] = jnp.broadcast_to(b_ref[...], o_ref.shape)

        o_ref[...] += lax.dot_general(
            a_ref[...], w_ref[...], (((1,), (1,)), ((), ())),
            preferred_element_type=jnp.float32)

    return pl.pallas_call(
        body,
        grid=(g,),
        in_specs=[
            pl.BlockSpec((8, kc), lambda i: (0, i)),
            pl.BlockSpec((LAT, kc), lambda i: (0, i)),
            pl.BlockSpec((1, LAT), lambda i: (0, 0)),
        ],
        out_specs=pl.BlockSpec((8, LAT), lambda i: (0, 0)),
        out_shape=jax.ShapeDtypeStruct((8, LAT), jnp.float32),
    )(a8, wl, bl.reshape(1, LAT))


def _lat_dec(z8, wd, bd):
    rc = 3200
    g = wd.shape[0] // rc

    def body(z_ref, w_ref, b_ref, o_ref):
        o_ref[...] = lax.dot_general(
            z_ref[...], w_ref[...], (((1,), (1,)), ((), ())),
            preferred_element_type=jnp.float32) + b_ref[...]

    return pl.pallas_call(
        body,
        grid=(g,),
        in_specs=[
            pl.BlockSpec((8, LAT), lambda i: (0, 0)),
            pl.BlockSpec((rc, LAT), lambda i: (i, 0)),
            pl.BlockSpec((1, rc), lambda i: (0, i)),
        ],
        out_specs=pl.BlockSpec((8, rc), lambda i: (0, i)),
        out_shape=jax.ShapeDtypeStruct((8, wd.shape[0]), jnp.float32),
    )(z8, wd, bd.reshape(1, wd.shape[0]))


def _expand_w(w, cin, cout, ctab):
    t = jnp.transpose(w.reshape(cout, L, cin), (1, 2, 0))
    j = jnp.arange(ctab)
    tl = t[:, j % cin, :]
    onehot = ((j[:, None] // cin) == jnp.arange(B)[None, :]) & (j < B * cin)[:, None]
    w4 = tl[:, :, None, :] * onehot[None, :, :, None].astype(w.dtype)
    return w4.reshape(L * ctab, B * cout)


def _padc(w, to):
    return jnp.pad(w, ((0, 0), (0, to - w.shape[1])))


def kernel(x, idx0, idx1, dt0_row, dt0_col, dt0_val, dt1_row, dt1_col, dt1_val,
           ut0_row, ut0_col, ut0_val, ut1_row, ut1_col, ut1_val,
           W_en0, b_en0, W_en1, b_en1, W_enl, b_enl, W_del, b_del,
           W_de1, b_de1, W_de2, b_de2, W_f, b_f):
    f32 = jnp.float32
    xt = jnp.transpose(x, (1, 0, 2)).reshape(N0, B * CIN)
    xt = jnp.pad(xt, ((0, 0), (0, 16 - B * CIN)))
    idx0f = jnp.pad(idx0.astype(jnp.int32).reshape(-1), (0, M0 - N0 * L))
    idx1f = jnp.pad(idx1.astype(jnp.int32).reshape(-1), (0, M1 - N1 * L))

    def colval(col, val, mp):
        c = jnp.pad(col.astype(jnp.int32), (0, mp - col.shape[0]))
        v = jnp.pad(val.astype(f32), (0, mp - val.shape[0]))
        return c, jnp.broadcast_to(v[:, None], (mp, 16))

    dt0c, dt0v = colval(dt0_col, dt0_val, N1P * K)
    dt1c, dt1v = colval(dt1_col, dt1_val, N2P * K)
    ut1c, ut1v = colval(ut1_col, ut1_val, N1P * K)
    ut0c, ut0v = colval(ut0_col, ut0_val, N0P * K)

    def pad128(b):
        return jnp.pad(jnp.tile(b, B), (0, 128 - B * b.shape[0]))

    g1 = _sc_gather(xt, idx0f, 16, 128, 90, 3, lin=True).reshape(N0P, L * 16)
    h1 = _tc_mm(g1, _expand_w(W_en0, CIN, C0, 16), jnp.tile(b_en0, B), True, 256)
    p1 = _sc_pool(h1, dt0c, dt0v, N1P, 64, 32, 10, 1, lin=True)
    g2 = _sc_gather(p1, idx1f, 64, 96, 30, 3, lin=True).reshape(N1P, L * 64)
    h2 = _tc_mm(g2, _expand_w(W_en1, C0, C1, 64), jnp.tile(b_en1, B), True, 256)
    p2 = _sc_pool(h2, dt1c, dt1v, N2P, 128, 40, 2, 1)

    p2t = jnp.transpose(p2[:N2].reshape(N2, B, C1), (1, 0, 2)).reshape(B, N2 * C1)
    z8 = _lat_enc(jnp.pad(p2t, ((0, 4), (0, 0))), W_enl, b_enl)
    d8 = _lat_dec(z8, W_del, b_del)
    d0 = jnp.transpose(d8[:B].reshape(B, N2, C1), (1, 0, 2)).reshape(N2, B * C1)

    p3 = _sc_pool(d0, ut1c, ut1v, N1P, 128, 32, 10, 1)
    g3 = _sc_gather(p3, idx1f, 128, 96, 30, 3).reshape(N1P, L * 128)
    h4 = _tc_mm(g3, _expand_w(W_de1, C1, C1, 128), jnp.tile(b_de1, B), True, 256)
    p4 = _sc_pool(h4, ut0c, ut0v, N0P, 128, 32, 40, 2)
    idx0f9 = idx0f + (jnp.arange(M0, dtype=jnp.int32) % L) * N0P
    t4 = _tc_mm_stack(p4, _expand_w(W_de2, C1, C0, 128), 256)
    h5 = _sc_sum9(t4, idx0f9, 64, jnp.tile(b_de2, B), True, 96, 120, 3)
    g5 = _sc_gather(h5, idx0f, 64, 128, 90, 3, lin=True).reshape(N0P, L * 64)
    wf = jnp.pad(_expand_w(W_f, C0, CIN, 64), ((0, 0), (0, 4)))
    bf = jnp.pad(jnp.tile(b_f, B), (0, 4))
    outp = _tc_mm(g5, wf, bf, False, 256)

    return jnp.transpose(outp[:N0, :B * CIN].reshape(N0, B, CIN), (1, 0, 2))
